# Initial kernel scaffold; baseline (speedup 1.0000x reference)
import jax
import jax.numpy as jnp
from jax.experimental import pallas as pl


def kernel(probas, labels):
    raise NotImplementedError("write your pallas kernel here")


# speedup vs baseline: 27.1621x; 27.1621x over previous
import jax
import jax.numpy as jnp
from jax import lax
from jax.experimental import pallas as pl
from jax.experimental.pallas import tpu as pltpu
from jax.experimental.pallas import tpu_sc as plsc

B, C, H, W = 4, 19, 224, 224
N = H * W
NUNITS = B * C
NC, NS, L = 2, 16, 16
NW = NC * NS
UPW = -(-NUNITS // NW)
NB = 2048
NCH = 8
CH = N // NCH
VPC = CH // L


def _sc_body(p_hbm, lab_hbm, out_hbm, p_v, lab_v, hist_v, res_v):
    w = lax.axis_index("s") * NC + lax.axis_index("c")

    for u in range(UPW):
        res_v[u] = jnp.zeros((L,), jnp.float32)

    def do_unit(u):
        unit = u * NW + w
        img = unit // C
        cls = unit % C

        def zero_body(j, _):
            hist_v[pl.ds(j * L, L)] = jnp.zeros((L,), jnp.int32)
            return 0

        lax.fori_loop(0, NB // L, zero_body, 0)

        for ch in range(NCH):
            pltpu.sync_copy(p_hbm.at[unit, pl.ds(ch * CH, CH)], p_v)
            pltpu.sync_copy(lab_hbm.at[img, pl.ds(ch * CH, CH)], lab_v)

            def vec_body(i, _):
                p = p_v[pl.ds(i * L, L)]
                lab = lab_v[pl.ds(i * L, L)]
                fg = lab == cls
                e = jnp.where(fg, p, 1.0 - p)
                k = jnp.minimum((e * float(NB)).astype(jnp.int32), NB - 1)
                val = jnp.where(fg, jnp.int32(65537), jnp.int32(1))
                plsc.addupdate_scatter(hist_v, [k], val)
                return 0

            lax.fori_loop(0, VPC, vec_body, 0)

        def tot_body(j, carry):
            ftot = carry
            v = hist_v[pl.ds(j * L, L)]
            cf = lax.shift_right_logical(v, 16)
            return ftot + jnp.sum(cf)

        ftot = lax.fori_loop(0, NB // L, tot_body, jnp.int32(0))
        g = ftot.astype(jnp.float32)

        def scan_body(j, carry):
            msuf, fsuf, jsum = carry
            jj = NB // L - 1 - j
            v = hist_v[pl.ds(jj * L, L)]
            ca = (v & 0xFFFF).astype(jnp.float32)
            cf = lax.shift_right_logical(v, 16).astype(jnp.float32)
            cum_a = plsc.cumsum(ca)
            cum_f = plsc.cumsum(cf)
            sa = jnp.sum(ca)
            sf = jnp.sum(cf)
            m = msuf + sa - cum_a + ca
            f = fsuf + sf - cum_f + cf
            union = g + m - f
            inter = g - f
            jac = 1.0 - inter / jnp.maximum(union, 1.0)
            return (msuf + sa, fsuf + sf, jsum + jnp.sum(jac))

        zero = jnp.float32(0.0)
        _, _, jsum = lax.fori_loop(0, NB // L, scan_body, (zero, zero, zero))
        loss = (jsum - 0.5) * jnp.float32(1.0 / NB)

        io = lax.broadcasted_iota(jnp.int32, (L,), 0)
        vec = jnp.where(io == 0, loss, jnp.where(io == 1, g, 0.0))
        res_v[u] = vec

    for u in range(UPW - 1):
        do_unit(u)

    @pl.when(w < NUNITS - (UPW - 1) * NW)
    def _():
        do_unit(UPW - 1)

    pltpu.sync_copy(res_v, out_hbm.at[w])


_hist_call = pl.kernel(
    _sc_body,
    out_type=jax.ShapeDtypeStruct((NW, UPW, L), jnp.float32),
    mesh=plsc.VectorSubcoreMesh(core_axis_name="c", subcore_axis_name="s"),
    compiler_params=pltpu.CompilerParams(needs_layout_passes=False),
    scratch_types=[
        pltpu.VMEM((CH,), jnp.float32),
        pltpu.VMEM((CH,), jnp.int32),
        pltpu.VMEM((NB,), jnp.int32),
        pltpu.VMEM((UPW, L), jnp.float32),
    ],
)


def kernel(probas, labels):
    p2 = probas.reshape(B * C, N)
    lab = labels.astype(jnp.int32).reshape(B, N)
    out = _hist_call(p2, lab)
    loss = out[:, :, 0].T.reshape(-1)[:NUNITS].reshape(B, C)
    g = out[:, :, 1].T.reshape(-1)[:NUNITS].reshape(B, C)
    mask = (g > 0).astype(jnp.float32)
    per_img = jnp.sum(loss * mask, axis=1) / jnp.sum(mask, axis=1)
    return jnp.mean(per_img)


# speedup vs baseline: 70.4213x; 2.5926x over previous
import jax
import jax.numpy as jnp
from jax import lax
from jax.experimental import pallas as pl
from jax.experimental.pallas import tpu as pltpu
from jax.experimental.pallas import tpu_sc as plsc

B, C, H, W = 4, 19, 224, 224
N = H * W
NUNITS = B * C
NC, NS, L = 2, 16, 16
NW = NC * NS
UPW = -(-NUNITS // NW)
NB = 2048
NCH = 8
CH = N // NCH
VPC = CH // L


def _sc_body(p_hbm, lab_hbm, out_hbm, p_v, lab_v, hist_v, res_v, sem_p, sem_l):
    w = lax.axis_index("s") * NC + lax.axis_index("c")

    for u in range(UPW):
        res_v[u] = jnp.zeros((L,), jnp.float32)

    def do_unit(u):
        unit = u * NW + w
        img = unit // C
        cls = unit % C

        def start(ch):
            buf = ch % 2
            cp = pltpu.async_copy(
                p_hbm.at[unit, pl.ds(ch * CH, CH)], p_v.at[buf], sem_p)
            cl = pltpu.async_copy(
                lab_hbm.at[img, pl.ds(ch * CH, CH)], lab_v.at[buf], sem_l)
            return cp, cl

        cur = start(0)

        @plsc.parallel_loop(0, NB // L, unroll=8)
        def _(j):
            hist_v[pl.ds(j * L, L)] = jnp.zeros((L,), jnp.int32)

        for ch in range(NCH):
            cp, cl = cur
            cp.wait()
            cl.wait()
            if ch + 1 < NCH:
                cur = start(ch + 1)
            buf = ch % 2

            @plsc.parallel_loop(0, VPC, unroll=8)
            def _(i):
                p = p_v[buf, pl.ds(i * L, L)]
                lab = lab_v[buf, pl.ds(i * L, L)]
                fg = lab == cls
                e = jnp.where(fg, p, 1.0 - p)
                k = jnp.minimum((e * float(NB)).astype(jnp.int32), NB - 1)
                val = jnp.where(fg, jnp.int32(65537), jnp.int32(1))
                plsc.addupdate_scatter(hist_v, [k], val)

        def tot_body(j, carry):
            ftot = carry
            v = hist_v[pl.ds(j * L, L)]
            cf = lax.shift_right_logical(v, 16)
            return ftot + jnp.sum(cf)

        ftot = lax.fori_loop(0, NB // L, tot_body, jnp.int32(0))
        g = ftot.astype(jnp.float32)

        def scan_body(j, carry):
            msuf, fsuf, jsum = carry
            jj = NB // L - 1 - j
            v = hist_v[pl.ds(jj * L, L)]
            ca = (v & 0xFFFF).astype(jnp.float32)
            cf = lax.shift_right_logical(v, 16).astype(jnp.float32)
            cum_a = plsc.cumsum(ca)
            cum_f = plsc.cumsum(cf)
            sa = jnp.sum(ca)
            sf = jnp.sum(cf)
            m = msuf + sa - cum_a + ca
            f = fsuf + sf - cum_f + cf
            union = g + m - f
            inter = g - f
            jac = 1.0 - inter / jnp.maximum(union, 1.0)
            return (msuf + sa, fsuf + sf, jsum + jnp.sum(jac))

        zero = jnp.float32(0.0)
        _, _, jsum = lax.fori_loop(0, NB // L, scan_body, (zero, zero, zero))
        loss = (jsum - 0.5) * jnp.float32(1.0 / NB)

        io = lax.broadcasted_iota(jnp.int32, (L,), 0)
        vec = jnp.where(io == 0, loss, jnp.where(io == 1, g, 0.0))
        res_v[u] = vec

    for u in range(UPW - 1):
        do_unit(u)

    @pl.when(w < NUNITS - (UPW - 1) * NW)
    def _():
        do_unit(UPW - 1)

    pltpu.sync_copy(res_v, out_hbm.at[w])


_hist_call = pl.kernel(
    _sc_body,
    out_type=jax.ShapeDtypeStruct((NW, UPW, L), jnp.float32),
    mesh=plsc.VectorSubcoreMesh(core_axis_name="c", subcore_axis_name="s"),
    compiler_params=pltpu.CompilerParams(needs_layout_passes=False),
    scratch_types=[
        pltpu.VMEM((2, CH), jnp.float32),
        pltpu.VMEM((2, CH), jnp.int32),
        pltpu.VMEM((NB,), jnp.int32),
        pltpu.VMEM((UPW, L), jnp.float32),
        pltpu.SemaphoreType.DMA,
        pltpu.SemaphoreType.DMA,
    ],
)


def kernel(probas, labels):
    p2 = probas.reshape(B * C, N)
    lab = labels.astype(jnp.int32).reshape(B, N)
    out = _hist_call(p2, lab)
    loss = out[:, :, 0].T.reshape(-1)[:NUNITS].reshape(B, C)
    g = out[:, :, 1].T.reshape(-1)[:NUNITS].reshape(B, C)
    mask = (g > 0).astype(jnp.float32)
    per_img = jnp.sum(loss * mask, axis=1) / jnp.sum(mask, axis=1)
    return jnp.mean(per_img)


# speedup vs baseline: 70.7582x; 1.0048x over previous
import jax
import jax.numpy as jnp
from jax import lax
from jax.experimental import pallas as pl
from jax.experimental.pallas import tpu as pltpu
from jax.experimental.pallas import tpu_sc as plsc

B, C, H, W = 4, 19, 224, 224
N = H * W
NUNITS = B * C
NC, NS, L = 2, 16, 16
NW = NC * NS
UPW = -(-NUNITS // NW)
NB = 1024
NCH = 8
CH = N // NCH
VPC = CH // L


def _sc_body(p_hbm, lab_hbm, out_hbm, p_v, lab_v, hist_v, res_v, sem_p, sem_l):
    w = lax.axis_index("s") * NC + lax.axis_index("c")
    ones = jnp.ones((L,), jnp.int32)

    for u in range(UPW):
        res_v[u] = jnp.zeros((L,), jnp.float32)

    def do_unit(u):
        unit = u * NW + w
        img = unit // C
        cls = unit % C

        def start(ch):
            buf = ch % 2
            cp = pltpu.async_copy(
                p_hbm.at[unit, pl.ds(ch * CH, CH)], p_v.at[buf], sem_p)
            cl = pltpu.async_copy(
                lab_hbm.at[img, pl.ds(ch * CH, CH)], lab_v.at[buf], sem_l)
            return cp, cl

        cur = start(0)

        @plsc.parallel_loop(0, 2 * NB // L, unroll=8)
        def _(j):
            hist_v[pl.ds(j * L, L)] = jnp.zeros((L,), jnp.int32)

        for ch in range(NCH):
            cp, cl = cur
            cp.wait()
            cl.wait()
            if ch + 1 < NCH:
                cur = start(ch + 1)
            buf = ch % 2

            @plsc.parallel_loop(0, VPC, unroll=8)
            def _(i):
                p = p_v[buf, pl.ds(i * L, L)]
                lab = lab_v[buf, pl.ds(i * L, L)]
                fg = lab == cls
                kp = (p * float(NB)).astype(jnp.int32)
                k2 = kp + jnp.where(fg, jnp.int32(0), jnp.int32(NB))
                plsc.addupdate_scatter(hist_v, [k2], ones)

        @plsc.parallel_loop(0, NB // L, unroll=4, carry=jnp.int32(0))
        def ftot(j, acc):
            return acc + jnp.sum(hist_v[pl.ds(j * L, L)])

        g = ftot.astype(jnp.float32)

        zero = jnp.float32(0.0)

        @plsc.parallel_loop(0, NB // L, unroll=4, carry=(zero, zero, zero))
        def scans(j, carry):
            msuf, fsuf, jsum = carry
            jj = NB // L - 1 - j
            cf = hist_v[pl.ds(jj * L, L)].astype(jnp.float32)
            cb = lax.rev(
                hist_v[pl.ds(2 * NB - L * (jj + 1), L)], (0,)
            ).astype(jnp.float32)
            ca = cf + cb
            cum_a = plsc.cumsum(ca)
            cum_f = plsc.cumsum(cf)
            sa = jnp.sum(ca)
            sf = jnp.sum(cf)
            m = msuf + sa - cum_a + ca
            f = fsuf + sf - cum_f + cf
            union = g + m - f
            inter = g - f
            jac = 1.0 - inter / jnp.maximum(union, 1.0)
            return (msuf + sa, fsuf + sf, jsum + jnp.sum(jac))

        _, _, jsum = scans
        loss = (jsum - 0.5) * jnp.float32(1.0 / NB)

        io = lax.broadcasted_iota(jnp.int32, (L,), 0)
        vec = jnp.where(io == 0, loss, jnp.where(io == 1, g, 0.0))
        res_v[u] = vec

    for u in range(UPW - 1):
        do_unit(u)

    @pl.when(w < NUNITS - (UPW - 1) * NW)
    def _():
        do_unit(UPW - 1)

    pltpu.sync_copy(res_v, out_hbm.at[w])


_hist_call = pl.kernel(
    _sc_body,
    out_type=jax.ShapeDtypeStruct((NW, UPW, L), jnp.float32),
    mesh=plsc.VectorSubcoreMesh(core_axis_name="c", subcore_axis_name="s"),
    compiler_params=pltpu.CompilerParams(needs_layout_passes=False),
    scratch_types=[
        pltpu.VMEM((2, CH), jnp.float32),
        pltpu.VMEM((2, CH), jnp.int32),
        pltpu.VMEM((2 * NB,), jnp.int32),
        pltpu.VMEM((UPW, L), jnp.float32),
        pltpu.SemaphoreType.DMA,
        pltpu.SemaphoreType.DMA,
    ],
)


def kernel(probas, labels):
    p2 = probas.reshape(B * C, N)
    lab = labels.astype(jnp.int32).reshape(B, N)
    out = _hist_call(p2, lab)
    loss = out[:, :, 0].T.reshape(-1)[:NUNITS].reshape(B, C)
    g = out[:, :, 1].T.reshape(-1)[:NUNITS].reshape(B, C)
    mask = (g > 0).astype(jnp.float32)
    per_img = jnp.sum(loss * mask, axis=1) / jnp.sum(mask, axis=1)
    return jnp.mean(per_img)


# speedup vs baseline: 93.6590x; 1.3236x over previous
import jax
import jax.numpy as jnp
from jax import lax
from jax.experimental import pallas as pl
from jax.experimental.pallas import tpu as pltpu
from jax.experimental.pallas import tpu_sc as plsc

B, C, H, W = 4, 19, 224, 224
N = H * W
NUNITS = B * C
NC, NS, L = 2, 16, 16
NW = NC * NS
UPW = -(-NUNITS // NW)
NB = 1024
NCH = 7
RPC = H // NCH
CH = RPC * W
GPR = W // L


def _sc_body(p_hbm, lab_hbm, out_hbm, p_v, lab_v, hist_v, res_v, sem_p, sem_l):
    w = lax.axis_index("s") * NC + lax.axis_index("c")
    ones = jnp.ones((L,), jnp.int32)

    def do_unit(u):
        unit = u * NW + w
        img = unit // C
        cls = unit % C

        def start(ch):
            buf = ch % 2
            cp = pltpu.async_copy(
                p_hbm.at[img, cls, pl.ds(ch * RPC, RPC), :], p_v.at[buf],
                sem_p)
            cl = pltpu.async_copy(
                lab_hbm.at[img, pl.ds(ch * RPC, RPC), :], lab_v.at[buf],
                sem_l)
            return cp, cl

        cur = start(0)

        @plsc.parallel_loop(0, 2 * NB // L, unroll=8)
        def _(j):
            hist_v[pl.ds(j * L, L)] = jnp.zeros((L,), jnp.int32)

        for ch in range(NCH):
            cp, cl = cur
            cp.wait()
            cl.wait()
            if ch + 1 < NCH:
                cur = start(ch + 1)
            buf = ch % 2

            @plsc.parallel_loop(0, RPC)
            def _(i):
                for gg in range(GPR):
                    p = p_v[buf, i, pl.ds(gg * L, L)]
                    lab = lab_v[buf, i, pl.ds(gg * L, L)]
                    fg = lab == cls
                    kp = (p * float(NB)).astype(jnp.int32)
                    k2 = kp + jnp.where(fg, jnp.int32(0), jnp.int32(NB))
                    plsc.addupdate_scatter(hist_v, [k2], ones)

        @plsc.parallel_loop(0, NB // L, unroll=4, carry=jnp.int32(0))
        def ftot(j, acc):
            return acc + jnp.sum(hist_v[pl.ds(j * L, L)])

        g = ftot.astype(jnp.float32)

        zero = jnp.float32(0.0)

        @plsc.parallel_loop(0, NB // L, unroll=4, carry=(zero, zero, zero))
        def scans(j, carry):
            msuf, fsuf, jsum = carry
            jj = NB // L - 1 - j
            cf = hist_v[pl.ds(jj * L, L)].astype(jnp.float32)
            cb = lax.rev(
                hist_v[pl.ds(2 * NB - L * (jj + 1), L)], (0,)
            ).astype(jnp.float32)
            ca = cf + cb
            cum_a = plsc.cumsum(ca)
            cum_f = plsc.cumsum(cf)
            sa = jnp.sum(ca)
            sf = jnp.sum(cf)
            m = msuf + sa - cum_a + ca
            f = fsuf + sf - cum_f + cf
            union = g + m - f
            inter = g - f
            jac = 1.0 - inter / jnp.maximum(union, 1.0)
            return (msuf + sa, fsuf + sf, jsum + jnp.sum(jac))

        _, _, jsum = scans
        loss = (jsum - 0.5) * jnp.float32(1.0 / NB)

        io = lax.broadcasted_iota(jnp.int32, (L,), 0)
        vec = jnp.where(io == 0, loss, jnp.where(io == 1, g, 0.0))
        res_v[pl.ds(u * L, L)] = vec

    def unit_body(u, _):
        res_v[pl.ds(u * L, L)] = jnp.zeros((L,), jnp.float32)

        @pl.when(u * NW + w < NUNITS)
        def _():
            do_unit(u)

        return 0

    lax.fori_loop(0, UPW, unit_body, 0)

    pltpu.sync_copy(res_v, out_hbm.at[w])


_hist_call = pl.kernel(
    _sc_body,
    out_type=jax.ShapeDtypeStruct((NW, UPW * L), jnp.float32),
    mesh=plsc.VectorSubcoreMesh(core_axis_name="c", subcore_axis_name="s"),
    compiler_params=pltpu.CompilerParams(
        needs_layout_passes=False, use_tc_tiling_on_sc=True),
    scratch_types=[
        pltpu.VMEM((2, RPC, W), jnp.float32),
        pltpu.VMEM((2, RPC, W), jnp.int32),
        pltpu.VMEM((2 * NB,), jnp.int32),
        pltpu.VMEM((UPW * L,), jnp.float32),
        pltpu.SemaphoreType.DMA,
        pltpu.SemaphoreType.DMA,
    ],
)


def kernel(probas, labels):
    lab = labels.astype(jnp.int32)
    out = _hist_call(probas, lab).reshape(NW, UPW, L)
    loss = out[:, :, 0].T.reshape(-1)[:NUNITS].reshape(B, C)
    g = out[:, :, 1].T.reshape(-1)[:NUNITS].reshape(B, C)
    mask = (g > 0).astype(jnp.float32)
    per_img = jnp.sum(loss * mask, axis=1) / jnp.sum(mask, axis=1)
    return jnp.mean(per_img)
